# Initial kernel scaffold; baseline (speedup 1.0000x reference)
import functools

import jax
import jax.numpy as jnp
from jax import lax
from jax.experimental import pallas as pl
from jax.experimental.pallas import tpu as pltpu
from jax.experimental.pallas import tpu_sc as plsc

N = 10000
F = 128
E = 320000

NC = 2
NS = 16
NW = NC * NS
CH = 128
NCHUNK = 79
EW = CH * NCHUNK
E_PAD = EW * NW
NP = 10240
RPT = NP // NS

_mesh = plsc.VectorSubcoreMesh(core_axis_name="c", subcore_axis_name="s")


@functools.partial(
    pl.kernel,
    out_type=jax.ShapeDtypeStruct((NC, NP), jnp.float32),
    mesh=_mesh,
    scratch_types=[
        pltpu.VMEM((RPT,), jnp.float32),
        pltpu.VMEM((CH,), jnp.int32),
        pltpu.VMEM((CH,), jnp.float32),
        pltpu.VMEM_SHARED((NP,), jnp.float32),
    ],
)
def _deg_kernel(col_hbm, ew_hbm, out_hbm, zb, col_v, ew_v, deg_sh):
    cid = lax.axis_index("c")
    sid = lax.axis_index("s")
    wid = cid * NS + sid

    @pl.loop(0, RPT, step=16)
    def _(i):
        zb[pl.ds(i, 16)] = jnp.zeros((16,), jnp.float32)

    pltpu.sync_copy(zb, deg_sh.at[pl.ds(sid * RPT, RPT)])
    plsc.subcore_barrier()

    base = wid * EW

    @pl.loop(0, NCHUNK)
    def _(j):
        off = base + j * CH
        pltpu.sync_copy(col_hbm.at[pl.ds(off, CH)], col_v)
        pltpu.sync_copy(ew_hbm.at[pl.ds(off, CH)], ew_v)
        pltpu.sync_copy(ew_v, deg_sh.at[col_v], add=True)

    plsc.subcore_barrier()
    pltpu.sync_copy(
        deg_sh.at[pl.ds(sid * RPT, RPT)],
        out_hbm.at[cid, pl.ds(sid * RPT, RPT)],
    )


@functools.partial(
    pl.kernel,
    out_type=jax.ShapeDtypeStruct((NC, NP, F), jnp.float32),
    mesh=_mesh,
    scratch_types=[
        pltpu.VMEM((CH, F), jnp.float32),
        pltpu.VMEM((CH,), jnp.int32),
        pltpu.VMEM((CH,), jnp.int32),
        pltpu.SMEM((CH,), jnp.float32),
        pltpu.VMEM_SHARED((NP, F), jnp.float32),
        pltpu.SemaphoreType.DMA,
    ],
)
def _agg_kernel(xs_hbm, row_hbm, col_hbm, ew_hbm, out_hbm,
                rows_v, row_v, col_v, ew_s, acc_sh, sem):
    cid = lax.axis_index("c")
    sid = lax.axis_index("s")
    wid = cid * NS + sid

    @pl.loop(0, CH)
    def _(r):
        for j in range(F // 16):
            rows_v[r, pl.ds(j * 16, 16)] = jnp.zeros((16,), jnp.float32)

    for k in range(RPT // CH):
        pltpu.sync_copy(rows_v, acc_sh.at[pl.ds(sid * RPT + k * CH, CH)])
    plsc.subcore_barrier()

    base = wid * EW

    @pl.loop(0, NCHUNK)
    def _(j):
        off = base + j * CH
        pltpu.sync_copy(row_hbm.at[pl.ds(off, CH)], row_v)
        pltpu.sync_copy(col_hbm.at[pl.ds(off, CH)], col_v)
        pltpu.sync_copy(ew_hbm.at[pl.ds(off, CH)], ew_s)
        pltpu.async_copy(xs_hbm.at[row_v], rows_v, sem).wait()

        @pl.loop(0, CH)
        def _(e):
            w = ew_s[e]
            for j2 in range(F // 16):
                rows_v[e, pl.ds(j2 * 16, 16)] = rows_v[e, pl.ds(j2 * 16, 16)] * w

        pltpu.sync_copy(rows_v, acc_sh.at[col_v], add=True)

    plsc.subcore_barrier()
    pltpu.sync_copy(
        acc_sh.at[pl.ds(sid * RPT, RPT)],
        out_hbm.at[cid, pl.ds(sid * RPT, RPT)],
    )


def _mm_body(x_ref, w_ref, o_ref):
    o_ref[...] = lax.dot_general(
        x_ref[...], w_ref[...], (((1,), (1,)), ((), ())),
        preferred_element_type=jnp.float32)


_mm = pl.pallas_call(
    _mm_body, out_shape=jax.ShapeDtypeStruct((N, F), jnp.float32))


def _scale_body(dp_ref, xp_ref, dinvb_ref, xs_ref):
    deg = dp_ref[0] + dp_ref[1] + 1.0
    dinv = jnp.where(deg > 0, lax.rsqrt(deg), 0.0)
    dinvb = jnp.broadcast_to(dinv, (N, F))
    dinvb_ref[...] = dinvb
    xs_ref[...] = dinvb * xp_ref[...]


_scale = pl.pallas_call(
    _scale_body,
    out_shape=(jax.ShapeDtypeStruct((N, F), jnp.float32),
               jax.ShapeDtypeStruct((N, F), jnp.float32)))


def _mid_body(a0_ref, a1_ref, xs1_ref, dinvb_ref, b1_ref, g_ref, be_ref,
              w2_ref, xs2_ref):
    dinvb = dinvb_ref[...]
    pre = dinvb * (a0_ref[...] + a1_ref[...] + xs1_ref[...]) + b1_ref[...]
    mean = jnp.mean(pre, axis=0, keepdims=True)
    cen = pre - mean
    var = jnp.mean(cen * cen, axis=0, keepdims=True)
    h = cen * lax.rsqrt(var + 1e-5) * g_ref[...] + be_ref[...]
    h = jnp.maximum(h, 0.0)
    xp2 = lax.dot_general(
        h, w2_ref[...], (((1,), (1,)), ((), ())),
        preferred_element_type=jnp.float32)
    xs2_ref[...] = dinvb * xp2


_mid = pl.pallas_call(
    _mid_body, out_shape=jax.ShapeDtypeStruct((N, F), jnp.float32))


def _out_body(a0_ref, a1_ref, xs2_ref, dinvb_ref, b2_ref, o_ref):
    o_ref[...] = (dinvb_ref[...] * (a0_ref[...] + a1_ref[...] + xs2_ref[...])
                  + b2_ref[...])


_out = pl.pallas_call(
    _out_body, out_shape=jax.ShapeDtypeStruct((N, F), jnp.float32))


def kernel(x, edge_index, edge_weight, W1, b1, gamma, beta, W2, b2):
    ei = edge_index.astype(jnp.int32)
    row, col = ei[0], ei[1]
    pad = E_PAD - E
    pad_idx = jnp.arange(pad, dtype=jnp.int32) % N
    row_p = jnp.concatenate([row, pad_idx])
    col_p = jnp.concatenate([col, pad_idx])
    ew_p = jnp.concatenate([edge_weight.astype(jnp.float32),
                            jnp.zeros((pad,), jnp.float32)])

    deg_parts = _deg_kernel(col_p, ew_p)
    xp1 = _mm(x, W1)
    dp = deg_parts[:, :N].reshape(NC, N, 1)
    dinvb, xs1 = _scale(dp, xp1)
    acc1 = _agg_kernel(xs1, row_p, col_p, ew_p)
    xs2 = _mid(acc1[0, :N], acc1[1, :N], xs1, dinvb,
               b1.reshape(1, F), gamma.reshape(1, F), beta.reshape(1, F),
               W2)
    acc2 = _agg_kernel(xs2, row_p, col_p, ew_p)
    return _out(acc2[0, :N], acc2[1, :N], xs2, dinvb, b2.reshape(1, F))


# speedup vs baseline: 12.3212x; 12.3212x over previous
import functools

import jax
import jax.numpy as jnp
from jax import lax
from jax.experimental import pallas as pl
from jax.experimental.pallas import tpu as pltpu
from jax.experimental.pallas import tpu_sc as plsc

N = 10000
F = 128
E = 320000

NC = 2
NS = 16
NW = NC * NS
CH = 128
NCHUNK = 79
EW = CH * NCHUNK
E_PAD = EW * NW
NP = 10240
RPT = NP // NS

_mesh = plsc.VectorSubcoreMesh(core_axis_name="c", subcore_axis_name="s")


@functools.partial(
    pl.kernel,
    out_type=jax.ShapeDtypeStruct((NC, NP), jnp.float32),
    mesh=_mesh,
    scratch_types=[
        pltpu.VMEM((RPT,), jnp.float32),
        pltpu.VMEM((CH,), jnp.int32),
        pltpu.VMEM((CH,), jnp.float32),
        pltpu.VMEM_SHARED((NP,), jnp.float32),
    ],
)
def _deg_kernel(col_hbm, ew_hbm, out_hbm, zb, col_v, ew_v, deg_sh):
    cid = lax.axis_index("c")
    sid = lax.axis_index("s")
    wid = cid * NS + sid

    @pl.loop(0, RPT, step=16)
    def _(i):
        zb[pl.ds(i, 16)] = jnp.zeros((16,), jnp.float32)

    pltpu.sync_copy(zb, deg_sh.at[pl.ds(sid * RPT, RPT)])
    plsc.subcore_barrier()

    base = wid * EW

    @pl.loop(0, NCHUNK)
    def _(j):
        off = base + j * CH
        pltpu.sync_copy(col_hbm.at[pl.ds(off, CH)], col_v)
        pltpu.sync_copy(ew_hbm.at[pl.ds(off, CH)], ew_v)
        pltpu.sync_copy(ew_v, deg_sh.at[col_v], add=True)

    plsc.subcore_barrier()
    pltpu.sync_copy(
        deg_sh.at[pl.ds(sid * RPT, RPT)],
        out_hbm.at[cid, pl.ds(sid * RPT, RPT)],
    )


@functools.partial(
    pl.kernel,
    out_type=jax.ShapeDtypeStruct((NC, NP, F), jnp.float32),
    mesh=_mesh,
    scratch_types=[
        pltpu.VMEM((CH, F), jnp.float32),
        pltpu.VMEM((CH,), jnp.int32),
        pltpu.VMEM((CH,), jnp.int32),
        pltpu.VMEM((CH,), jnp.float32),
        pltpu.VMEM_SHARED((NP, F), jnp.float32),
        pltpu.SemaphoreType.DMA,
    ],
)
def _agg_kernel(xs_hbm, row_hbm, col_hbm, ew_hbm, out_hbm,
                rows_v, row_v, col_v, ew_v, acc_sh, sem):
    cid = lax.axis_index("c")
    sid = lax.axis_index("s")
    wid = cid * NS + sid

    @pl.loop(0, CH)
    def _(r):
        for j in range(F // 16):
            rows_v[r, pl.ds(j * 16, 16)] = jnp.zeros((16,), jnp.float32)

    for k in range(RPT // CH):
        pltpu.sync_copy(rows_v, acc_sh.at[pl.ds(sid * RPT + k * CH, CH)])
    plsc.subcore_barrier()

    base = wid * EW

    @pl.loop(0, NCHUNK)
    def _(j):
        off = base + j * CH
        pltpu.sync_copy(row_hbm.at[pl.ds(off, CH)], row_v)
        pltpu.sync_copy(col_hbm.at[pl.ds(off, CH)], col_v)
        pltpu.sync_copy(ew_hbm.at[pl.ds(off, CH)], ew_v)
        pltpu.async_copy(xs_hbm.at[row_v], rows_v, sem).wait()

        @pl.loop(0, CH, step=16)
        def _(g):
            ew16 = ew_v[pl.ds(g, 16)]
            for e16 in range(16):
                w = lax.gather(
                    ew16, jnp.full((16, 1), e16, jnp.int32),
                    lax.GatherDimensionNumbers(
                        offset_dims=(), collapsed_slice_dims=(0,),
                        start_index_map=(0,)),
                    (1,), mode=lax.GatherScatterMode.PROMISE_IN_BOUNDS)
                for j2 in range(F // 16):
                    rows_v[g + e16, pl.ds(j2 * 16, 16)] = (
                        rows_v[g + e16, pl.ds(j2 * 16, 16)] * w)

        pltpu.sync_copy(rows_v, acc_sh.at[col_v], add=True)

    plsc.subcore_barrier()
    pltpu.sync_copy(
        acc_sh.at[pl.ds(sid * RPT, RPT)],
        out_hbm.at[cid, pl.ds(sid * RPT, RPT)],
    )


def _mm_body(x_ref, w_ref, o_ref):
    o_ref[...] = lax.dot_general(
        x_ref[...], w_ref[...], (((1,), (1,)), ((), ())),
        preferred_element_type=jnp.float32)


_mm = pl.pallas_call(
    _mm_body, out_shape=jax.ShapeDtypeStruct((N, F), jnp.float32))


def _scale_body(dp_ref, xp_ref, dinvb_ref, xs_ref):
    deg = dp_ref[0] + dp_ref[1] + 1.0
    dinv = jnp.where(deg > 0, lax.rsqrt(deg), 0.0)
    dinvb = jnp.broadcast_to(dinv, (N, F))
    dinvb_ref[...] = dinvb
    xs_ref[...] = dinvb * xp_ref[...]


_scale = pl.pallas_call(
    _scale_body,
    out_shape=(jax.ShapeDtypeStruct((N, F), jnp.float32),
               jax.ShapeDtypeStruct((N, F), jnp.float32)))


def _mid_body(a0_ref, a1_ref, xs1_ref, dinvb_ref, b1_ref, g_ref, be_ref,
              w2_ref, xs2_ref):
    dinvb = dinvb_ref[...]
    pre = dinvb * (a0_ref[...] + a1_ref[...] + xs1_ref[...]) + b1_ref[...]
    mean = jnp.mean(pre, axis=0, keepdims=True)
    cen = pre - mean
    var = jnp.mean(cen * cen, axis=0, keepdims=True)
    h = cen * lax.rsqrt(var + 1e-5) * g_ref[...] + be_ref[...]
    h = jnp.maximum(h, 0.0)
    xp2 = lax.dot_general(
        h, w2_ref[...], (((1,), (1,)), ((), ())),
        preferred_element_type=jnp.float32)
    xs2_ref[...] = dinvb * xp2


_mid = pl.pallas_call(
    _mid_body, out_shape=jax.ShapeDtypeStruct((N, F), jnp.float32))


def _out_body(a0_ref, a1_ref, xs2_ref, dinvb_ref, b2_ref, o_ref):
    o_ref[...] = (dinvb_ref[...] * (a0_ref[...] + a1_ref[...] + xs2_ref[...])
                  + b2_ref[...])


_out = pl.pallas_call(
    _out_body, out_shape=jax.ShapeDtypeStruct((N, F), jnp.float32))


def kernel(x, edge_index, edge_weight, W1, b1, gamma, beta, W2, b2):
    ei = edge_index.astype(jnp.int32)
    row, col = ei[0], ei[1]
    pad = E_PAD - E
    pad_idx = jnp.arange(pad, dtype=jnp.int32) % N
    row_p = jnp.concatenate([row, pad_idx])
    col_p = jnp.concatenate([col, pad_idx])
    ew_p = jnp.concatenate([edge_weight.astype(jnp.float32),
                            jnp.zeros((pad,), jnp.float32)])

    deg_parts = _deg_kernel(col_p, ew_p)
    xp1 = _mm(x, W1)
    dp = deg_parts[:, :N].reshape(NC, N, 1)
    dinvb, xs1 = _scale(dp, xp1)
    acc1 = _agg_kernel(xs1, row_p, col_p, ew_p)
    xs2 = _mid(acc1[0, :N], acc1[1, :N], xs1, dinvb,
               b1.reshape(1, F), gamma.reshape(1, F), beta.reshape(1, F),
               W2)
    acc2 = _agg_kernel(xs2, row_p, col_p, ew_p)
    return _out(acc2[0, :N], acc2[1, :N], xs2, dinvb, b2.reshape(1, F))
